# Initial kernel scaffold; baseline (speedup 1.0000x reference)
import jax
import jax.numpy as jnp
from jax.experimental import pallas as pl


def kernel(variance_map, segmentation_map):
    raise NotImplementedError("write your pallas kernel here")


# speedup vs baseline: 1.2609x; 1.2609x over previous
import jax
import jax.numpy as jnp
from jax.experimental import pallas as pl
from jax.experimental.pallas import tpu as pltpu

SEG_THR = 0.7
GAUSS_THR = 0.7
EPS = 1e-07
PI = 3.14
BIG = 1e30
H = 80
W = 80
N = H * W
SHIFT = 40.0
OT = 640
CT = 640


def _coef_kernel(var_ref, seg_ref, ut_ref):
    v0 = var_ref[0, 0:1, :]
    v1 = var_ref[0, 1:2, :]
    v2 = var_ref[0, 2:3, :]
    var_h = jnp.maximum(v0, 0.0) + 1.0
    var_w = jnp.maximum(v1, 0.0) + 1.0
    theta = PI * jax.nn.sigmoid(v2)
    s = jnp.sin(theta)
    co = jnp.cos(theta)
    vh2 = var_h * var_h
    vw2 = var_w * var_w
    a = co * co / (2.0 * vh2) + s * s / (2.0 * vw2)
    b = -2.0 * s * co / (4.0 * vh2) + 2.0 * s * co / (4.0 * vw2)
    c = s * s / (2.0 * vh2) + co * co / (2.0 * vw2)
    tb = 2.0 * b

    idx = jax.lax.broadcasted_iota(jnp.int32, (1, N), 1).astype(jnp.float32)
    x = jnp.floor((idx + 0.5) * (1.0 / W))
    y = idx - W * x
    X = x - SHIFT
    Y = y - SHIFT

    mask_pen = jnp.where(seg_ref[0, 0:1, :] > SEG_THR, 0.0, BIG)
    k0 = a
    k1 = -(2.0 * a * X + tb * Y)
    k2 = tb
    k3 = -(tb * X + 2.0 * c * Y)
    k4 = c
    k5 = a * X * X + tb * X * Y + c * Y * Y + mask_pen
    z = jnp.zeros_like(a)
    ut_ref[0] = jnp.concatenate([k0, k1, k2, k3, k4, k5, z, z], axis=0)


def _heatmap_kernel(ut_ref, out_ref):
    j = pl.program_id(1)

    o = (j * OT + jax.lax.broadcasted_iota(jnp.int32, (1, OT), 1)).astype(
        jnp.float32)
    p = jnp.floor((o + 0.5) * (1.0 / W))
    q = o - W * p
    P = p - SHIFT
    Q = q - SHIFT
    ones = jnp.ones_like(P)
    zeros = jnp.zeros_like(P)
    V = jnp.concatenate([P * P, P, P * Q, Q, Q * Q, ones, zeros, zeros],
                        axis=0)

    m = jnp.full((1, OT), BIG, dtype=jnp.float32)
    for c0 in range(0, N, CT):
        Us = ut_ref[0, :, c0:c0 + CT]
        E = jax.lax.dot_general(
            Us, V, (((0,), (0,)), ((), ())),
            preferred_element_type=jnp.float32,
            precision=jax.lax.Precision.HIGHEST)
        m = jnp.minimum(m, jnp.min(E, axis=0, keepdims=True))

    g = jnp.exp(-m + EPS)
    out_ref[0] = jnp.where(g >= GAUSS_THR, g, 0.0)


def kernel(variance_map, segmentation_map):
    B = variance_map.shape[0]
    var_flat = variance_map.reshape(B, 3, N)
    seg_flat = segmentation_map.reshape(B, 1, N)

    ut = pl.pallas_call(
        _coef_kernel,
        grid=(B,),
        in_specs=[
            pl.BlockSpec((1, 3, N), lambda b: (b, 0, 0)),
            pl.BlockSpec((1, 1, N), lambda b: (b, 0, 0)),
        ],
        out_specs=pl.BlockSpec((1, 8, N), lambda b: (b, 0, 0)),
        out_shape=jax.ShapeDtypeStruct((B, 8, N), jnp.float32),
        compiler_params=pltpu.CompilerParams(
            dimension_semantics=("parallel",)),
    )(var_flat, seg_flat)

    out = pl.pallas_call(
        _heatmap_kernel,
        grid=(B, N // OT),
        in_specs=[pl.BlockSpec((1, 8, N), lambda b, j: (b, 0, 0))],
        out_specs=pl.BlockSpec((1, 1, OT), lambda b, j: (b, 0, j)),
        out_shape=jax.ShapeDtypeStruct((B, 1, N), jnp.float32),
        compiler_params=pltpu.CompilerParams(
            dimension_semantics=("parallel", "arbitrary")),
    )(ut)

    return out.reshape(B, 1, H, W)


# speedup vs baseline: 3.4913x; 2.7689x over previous
import jax
import jax.numpy as jnp
from jax.experimental import pallas as pl
from jax.experimental.pallas import tpu as pltpu

SEG_THR = 0.7
GAUSS_THR = 0.7
EPS = 1e-07
PI = 3.14
BIG = 1e30
H = 80
W = 80
N = H * W
SHIFT = 40.0
OT = 640
CT = 640
NCHUNK = N // CT
ROWS_PER_CHUNK = CT // W
T_PRUNE = 0.36


def _coef_kernel(var_ref, seg_ref, ut_ref, lam_ref):
    b = pl.program_id(0)
    v0 = var_ref[0, 0:1, :]
    v1 = var_ref[0, 1:2, :]
    v2 = var_ref[0, 2:3, :]
    var_h = jnp.maximum(v0, 0.0) + 1.0
    var_w = jnp.maximum(v1, 0.0) + 1.0
    theta = PI * jax.nn.sigmoid(v2)
    s = jnp.sin(theta)
    co = jnp.cos(theta)
    vh2 = var_h * var_h
    vw2 = var_w * var_w
    a = co * co / (2.0 * vh2) + s * s / (2.0 * vw2)
    b_ = -2.0 * s * co / (4.0 * vh2) + 2.0 * s * co / (4.0 * vw2)
    c = s * s / (2.0 * vh2) + co * co / (2.0 * vw2)
    tb = 2.0 * b_

    idx = jax.lax.broadcasted_iota(jnp.int32, (1, N), 1).astype(jnp.float32)
    x = jnp.floor((idx + 0.5) * (1.0 / W))
    y = idx - W * x
    X = x - SHIFT
    Y = y - SHIFT

    mask_pen = jnp.where(seg_ref[0, 0:1, :] > SEG_THR, 0.0, BIG)
    k0 = a
    k1 = -(2.0 * a * X + tb * Y)
    k2 = tb
    k3 = -(tb * X + 2.0 * c * Y)
    k4 = c
    k5 = a * X * X + tb * X * Y + c * Y * Y + mask_pen
    z = jnp.zeros_like(a)
    ut_ref[0] = jnp.concatenate([k0, k1, k2, k3, k4, k5, z, z], axis=0)

    lam = jnp.minimum(1.0 / (2.0 * vh2), 1.0 / (2.0 * vw2))
    for k in range(NCHUNK):
        lam_ref[b, k] = jnp.min(lam[0:1, k * CT:(k + 1) * CT])


def _heatmap_kernel(ut_ref, lam_ref, out_ref, m_ref):
    b = pl.program_id(0)
    j = pl.program_id(1)

    o = (j * OT + jax.lax.broadcasted_iota(jnp.int32, (1, OT), 1)).astype(
        jnp.float32)
    p = jnp.floor((o + 0.5) * (1.0 / W))
    q = o - W * p
    P = p - SHIFT
    Q = q - SHIFT
    ones = jnp.ones_like(P)
    zeros = jnp.zeros_like(P)
    V = jnp.concatenate([P * P, P, P * Q, Q, Q * Q, ones, zeros, zeros],
                        axis=0)

    m_ref[...] = jnp.full((1, OT), BIG, dtype=jnp.float32)
    for k in range(NCHUNK):
        d = jnp.maximum(
            jnp.abs(j - k) * ROWS_PER_CHUNK - (ROWS_PER_CHUNK - 1), 0
        ).astype(jnp.float32)
        keep = lam_ref[b, k] * d * d <= T_PRUNE

        @pl.when(keep)
        def _(k=k):
            Us = ut_ref[0, :, k * CT:(k + 1) * CT]
            E = jax.lax.dot_general(
                Us, V, (((0,), (0,)), ((), ())),
                preferred_element_type=jnp.float32,
                precision=jax.lax.Precision.HIGHEST)
            m_ref[...] = jnp.minimum(m_ref[...],
                                     jnp.min(E, axis=0, keepdims=True))

    g = jnp.exp(-m_ref[...] + EPS)
    out_ref[0] = jnp.where(g >= GAUSS_THR, g, 0.0)


def kernel(variance_map, segmentation_map):
    B = variance_map.shape[0]
    var_flat = variance_map.reshape(B, 3, N)
    seg_flat = segmentation_map.reshape(B, 1, N)

    ut, lam = pl.pallas_call(
        _coef_kernel,
        grid=(B,),
        in_specs=[
            pl.BlockSpec((1, 3, N), lambda b: (b, 0, 0)),
            pl.BlockSpec((1, 1, N), lambda b: (b, 0, 0)),
        ],
        out_specs=[
            pl.BlockSpec((1, 8, N), lambda b: (b, 0, 0)),
            pl.BlockSpec(memory_space=pltpu.SMEM),
        ],
        out_shape=[
            jax.ShapeDtypeStruct((B, 8, N), jnp.float32),
            jax.ShapeDtypeStruct((B, NCHUNK), jnp.float32),
        ],
        compiler_params=pltpu.CompilerParams(
            dimension_semantics=("arbitrary",)),
    )(var_flat, seg_flat)

    out = pl.pallas_call(
        _heatmap_kernel,
        grid=(B, N // OT),
        in_specs=[
            pl.BlockSpec((1, 8, N), lambda b, j: (b, 0, 0)),
            pl.BlockSpec(memory_space=pltpu.SMEM),
        ],
        out_specs=pl.BlockSpec((1, 1, OT), lambda b, j: (b, 0, j)),
        out_shape=jax.ShapeDtypeStruct((B, 1, N), jnp.float32),
        scratch_shapes=[pltpu.VMEM((1, OT), jnp.float32)],
        compiler_params=pltpu.CompilerParams(
            dimension_semantics=("arbitrary", "arbitrary")),
    )(ut, lam)

    return out.reshape(B, 1, H, W)


# speedup vs baseline: 3.6267x; 1.0388x over previous
import jax
import jax.numpy as jnp
from jax.experimental import pallas as pl
from jax.experimental.pallas import tpu as pltpu

SEG_THR = 0.7
GAUSS_THR = 0.7
EPS = 1e-07
PI = 3.14
BIG = 1e30
H = 80
W = 80
N = H * W
SHIFT = 40.0
OT = 640
OT_ROWS = OT // W
CT = 320
NCHUNK = N // CT
CT_ROWS = CT // W
T_PRUNE = 0.36
T_LN = 0.35667494393873245
DELTA = 1e-3


def _coef_kernel(var_ref, seg_ref, ut_ref, lam_ref):
    b = pl.program_id(0)
    v0 = var_ref[0, 0:1, :]
    v1 = var_ref[0, 1:2, :]
    v2 = var_ref[0, 2:3, :]
    var_h = jnp.maximum(v0, 0.0) + 1.0
    var_w = jnp.maximum(v1, 0.0) + 1.0
    theta = PI * jax.nn.sigmoid(v2)
    s = jnp.sin(theta)
    co = jnp.cos(theta)
    vh2 = var_h * var_h
    vw2 = var_w * var_w
    a = co * co / (2.0 * vh2) + s * s / (2.0 * vw2)
    b_ = -2.0 * s * co / (4.0 * vh2) + 2.0 * s * co / (4.0 * vw2)
    c = s * s / (2.0 * vh2) + co * co / (2.0 * vw2)
    tb = 2.0 * b_

    idx = jax.lax.broadcasted_iota(jnp.int32, (1, N), 1).astype(jnp.float32)
    x = jnp.floor((idx + 0.5) * (1.0 / W))
    y = idx - W * x
    X = x - SHIFT
    Y = y - SHIFT

    mask_pen = jnp.where(seg_ref[0, 0:1, :] > SEG_THR, 0.0, BIG)
    k0 = a
    k1 = -(2.0 * a * X + tb * Y)
    k2 = tb
    k3 = -(tb * X + 2.0 * c * Y)
    k4 = c
    k5 = a * X * X + tb * X * Y + c * Y * Y + mask_pen
    z = jnp.zeros_like(a)
    one = jnp.ones_like(a)
    ut_ref[0] = jnp.concatenate(
        [k0, k1, k2, k3, k4, k5, z, z,
         X * X, X, X * Y, Y, Y * Y, one, z, z,
         a, tb, c, mask_pen, x, y, z, z], axis=0)

    lam = jnp.minimum(1.0 / (2.0 * vh2), 1.0 / (2.0 * vw2))
    for k in range(NCHUNK):
        lam_ref[b, k] = jnp.min(lam[0:1, k * CT:(k + 1) * CT])


def _heatmap_kernel(ut_ref, lam_ref, out_ref, m_ref):
    b = pl.program_id(0)
    j = pl.program_id(1)

    V = ut_ref[0, 8:16, pl.ds(pl.multiple_of(j * OT, OT), OT)]

    m_ref[...] = jnp.full((1, OT), BIG, dtype=jnp.float32)
    t0 = j * OT_ROWS
    for k in range(NCHUNK):
        c0 = k * CT_ROWS
        d = jnp.maximum(
            jnp.maximum(c0 - t0 - (OT_ROWS - 1), t0 - c0 - (CT_ROWS - 1)),
            0).astype(jnp.float32)
        keep = lam_ref[b, k] * d * d <= T_PRUNE

        @pl.when(keep)
        def _(k=k):
            Us = ut_ref[0, 0:8, k * CT:(k + 1) * CT]
            E = jax.lax.dot_general(
                Us, V, (((0,), (0,)), ((), ())),
                preferred_element_type=jnp.float32,
                precision=jax.lax.Precision.HIGHEST)
            m_ref[...] = jnp.minimum(m_ref[...],
                                     jnp.min(E, axis=0, keepdims=True))

    m = m_ref[...]
    lane = jax.lax.broadcasted_iota(jnp.int32, (1, OT), 1)
    band = jnp.logical_and(m > T_LN - DELTA, m < T_LN + DELTA)
    cand = jnp.where(band, lane, OT)
    arow = ut_ref[0, 16:17, :]
    tbrow = ut_ref[0, 17:18, :]
    crow = ut_ref[0, 18:19, :]
    penrow = ut_ref[0, 19:20, :]
    xrow = ut_ref[0, 20:21, :]
    yrow = ut_ref[0, 21:22, :]

    def refine_cond(idx):
        return idx < OT

    def refine_body(idx):
        o = (j * OT + idx).astype(jnp.float32)
        pf = jnp.floor((o + 0.5) * (1.0 / W))
        qf = o - W * pf
        dx = pf - xrow
        dy = qf - yrow
        Em = (arow * (dx * dx) + (tbrow * dx) * dy) + crow * (dy * dy)
        Em = Em + penrow
        me = jnp.min(Em)
        m_ref[...] = jnp.where(lane == idx, me, m_ref[...])
        return jnp.min(jnp.where(lane > idx, cand, OT))

    jax.lax.while_loop(refine_cond, refine_body, jnp.min(cand))

    g = jnp.exp(-m_ref[...] + EPS)
    out_ref[0] = jnp.where(g >= GAUSS_THR, g, 0.0)


def kernel(variance_map, segmentation_map):
    B = variance_map.shape[0]
    var_flat = variance_map.reshape(B, 3, N)
    seg_flat = segmentation_map.reshape(B, 1, N)

    ut, lam = pl.pallas_call(
        _coef_kernel,
        grid=(B,),
        in_specs=[
            pl.BlockSpec((1, 3, N), lambda b: (b, 0, 0)),
            pl.BlockSpec((1, 1, N), lambda b: (b, 0, 0)),
        ],
        out_specs=[
            pl.BlockSpec((1, 24, N), lambda b: (b, 0, 0)),
            pl.BlockSpec(memory_space=pltpu.SMEM),
        ],
        out_shape=[
            jax.ShapeDtypeStruct((B, 24, N), jnp.float32),
            jax.ShapeDtypeStruct((B, NCHUNK), jnp.float32),
        ],
        compiler_params=pltpu.CompilerParams(
            dimension_semantics=("arbitrary",)),
    )(var_flat, seg_flat)

    out = pl.pallas_call(
        _heatmap_kernel,
        grid=(B, N // OT),
        in_specs=[
            pl.BlockSpec((1, 24, N), lambda b, j: (b, 0, 0)),
            pl.BlockSpec(memory_space=pltpu.SMEM),
        ],
        out_specs=pl.BlockSpec((1, 1, OT), lambda b, j: (b, 0, j)),
        out_shape=jax.ShapeDtypeStruct((B, 1, N), jnp.float32),
        scratch_shapes=[pltpu.VMEM((1, OT), jnp.float32)],
        compiler_params=pltpu.CompilerParams(
            dimension_semantics=("arbitrary", "arbitrary")),
    )(ut, lam)

    return out.reshape(B, 1, H, W)
